# baseline (speedup 1.0000x reference)
import jax
import jax.numpy as jnp
from jax.experimental import pallas as pl
from jax.experimental.pallas import tpu as pltpu

B = 8; S = 8; NT = 4; T = 128; D = 64; K = 256; V = 20000
VT = 10112
NC = 1
LOG2E = 1.4426950408889634


def _body(subj_ref, task_ref, xp, fmu, fsig, smu, ssig, tmu, tsig,
          epsF, epsP, epsS, wc0, wc1, wc2, ww, wtop, wbot, y_ref,
          caug_s, wts_s):
    v = pl.program_id(1)

    @pl.when(v == 0)
    def _precompute():
        f32 = jnp.float32
        z_f = fmu[0] + fsig[0] * epsF[0]
        z_p = smu[0] + ssig[0] * epsP[0]
        z_s = tmu[0] + tsig[0] * epsS[0]

        c0 = jnp.dot(z_f, wc0[...], preferred_element_type=f32)
        c1 = jnp.dot(z_f, wc1[...], preferred_element_type=f32)
        c2 = jnp.dot(z_f, wc2[...], preferred_element_type=f32)
        logw = jnp.dot(z_f, ww[...], preferred_element_type=f32) + 2.0
        invw = jnp.exp(-logw)

        iwl = LOG2E * invw
        iw2 = 2.0 * iwl
        cenw = -iwl * (c0 * c0 + c1 * c1 + c2 * c2)
        zrow3 = jnp.zeros((3, K), dtype=f32)
        caug8 = jnp.concatenate(
            [iw2 * c0, iw2 * c1, iw2 * c2, -iwl, cenw, zrow3], axis=0)
        caug_s[...] = caug8.T.astype(jnp.bfloat16)

        wts = (jnp.dot(z_s, wbot[...], preferred_element_type=f32)
               + jnp.dot(z_p, wtop[...], preferred_element_type=f32))
        wts_s[...] = wts.astype(jnp.bfloat16)

    CS = VT // NC
    for i in range(NC):
        sl = pl.ds(i * CS, CS)
        arg = jnp.dot(caug_s[...], xp[:, sl],
                      preferred_element_type=jnp.float32)
        f = jnp.exp2(arg.astype(jnp.bfloat16))
        y_ref[0, :, sl] = jnp.dot(wts_s[...], f,
                                  preferred_element_type=jnp.float32)


def kernel(locations, block_subjects, block_tasks, factors_mu, factors_sigma,
           subject_mu, subject_sigma, task_mu, task_sigma, eps_F, eps_P,
           eps_S, W_c, W_w, W_wt):
    xaug = jnp.concatenate(
        [locations,
         jnp.sum(locations * locations, axis=1, keepdims=True),
         jnp.ones((V, 1), jnp.float32),
         jnp.zeros((V, 3), jnp.float32)], axis=1)
    xaugT = xaug.T.astype(jnp.bfloat16)

    wc = W_c.reshape(D, K, 3)
    wc0 = wc[:, :, 0]
    wc1 = wc[:, :, 1]
    wc2 = wc[:, :, 2]
    wtop = W_wt[:D]
    wbot = W_wt[D:]
    fmu3 = factors_mu[:, None, :]
    fsig3 = factors_sigma[:, None, :]
    smu3 = subject_mu[:, None, :]
    ssig3 = subject_sigma[:, None, :]
    epsF3 = eps_F[:, None, :]
    epsP3 = eps_P[:, None, :]

    nv = pl.cdiv(V, VT)
    grid_spec = pltpu.PrefetchScalarGridSpec(
        num_scalar_prefetch=2,
        grid=(B, nv),
        in_specs=[
            pl.BlockSpec((8, VT), lambda b, v, s, t: (0, v)),
            pl.BlockSpec((1, 1, D), lambda b, v, s, t: (s[b], 0, 0)),
            pl.BlockSpec((1, 1, D), lambda b, v, s, t: (s[b], 0, 0)),
            pl.BlockSpec((1, 1, D), lambda b, v, s, t: (s[b], 0, 0)),
            pl.BlockSpec((1, 1, D), lambda b, v, s, t: (s[b], 0, 0)),
            pl.BlockSpec((1, T, D), lambda b, v, s, t: (t[b], 0, 0)),
            pl.BlockSpec((1, T, D), lambda b, v, s, t: (t[b], 0, 0)),
            pl.BlockSpec((1, 1, D), lambda b, v, s, t: (b, 0, 0)),
            pl.BlockSpec((1, 1, D), lambda b, v, s, t: (b, 0, 0)),
            pl.BlockSpec((1, T, D), lambda b, v, s, t: (b, 0, 0)),
            pl.BlockSpec((D, K), lambda b, v, s, t: (0, 0)),
            pl.BlockSpec((D, K), lambda b, v, s, t: (0, 0)),
            pl.BlockSpec((D, K), lambda b, v, s, t: (0, 0)),
            pl.BlockSpec((D, K), lambda b, v, s, t: (0, 0)),
            pl.BlockSpec((D, K), lambda b, v, s, t: (0, 0)),
            pl.BlockSpec((D, K), lambda b, v, s, t: (0, 0)),
        ],
        out_specs=pl.BlockSpec((1, T, VT), lambda b, v, s, t: (b, 0, v)),
        scratch_shapes=[
            pltpu.VMEM((K, 8), jnp.bfloat16),
            pltpu.VMEM((T, K), jnp.bfloat16),
        ],
    )
    y = pl.pallas_call(
        _body,
        grid_spec=grid_spec,
        out_shape=jax.ShapeDtypeStruct((B, T, V), jnp.float32),
        compiler_params=pltpu.CompilerParams(
            dimension_semantics=("parallel", "arbitrary"),
        ),
    )(block_subjects, block_tasks, xaugT, fmu3, fsig3, smu3, ssig3,
      task_mu, task_sigma, epsF3, epsP3, eps_S, wc0, wc1, wc2, W_w,
      wtop, wbot)
    return y


# speedup vs baseline: 1.8790x; 1.8790x over previous
import jax
import jax.numpy as jnp
from jax.experimental import pallas as pl
from jax.experimental.pallas import tpu as pltpu

B = 8; S = 8; NT = 4; T = 128; D = 64; K = 256; V = 20000
VT = 10112
LOG2E = 1.4426950408889634


def _body(subj_ref, task_ref, xp, fmu, fsig, smu, ssig, tmu, tsig,
          epsF, epsP, epsS, wc0, wc1, wc2, ww, wtopT, wbotT, y_ref,
          caugT_s, wtsT_s):
    v = pl.program_id(1)

    @pl.when(v == 0)
    def _precompute():
        f32 = jnp.float32
        z_f = fmu[0] + fsig[0] * epsF[0]
        z_p = smu[0] + ssig[0] * epsP[0]
        z_s = tmu[0] + tsig[0] * epsS[0]

        c0 = jnp.dot(z_f, wc0[...], preferred_element_type=f32)
        c1 = jnp.dot(z_f, wc1[...], preferred_element_type=f32)
        c2 = jnp.dot(z_f, wc2[...], preferred_element_type=f32)
        logw = jnp.dot(z_f, ww[...], preferred_element_type=f32) + 2.0
        invw = jnp.exp(-logw)

        iwl = LOG2E * invw
        iw2 = 2.0 * iwl
        cenw = -iwl * (c0 * c0 + c1 * c1 + c2 * c2)
        zrow3 = jnp.zeros((3, K), dtype=f32)
        caugT_s[...] = jnp.concatenate(
            [iw2 * c0, iw2 * c1, iw2 * c2, -iwl, cenw, zrow3],
            axis=0).astype(jnp.bfloat16)

        i0 = jax.lax.broadcasted_iota(jnp.int32, (D, D), 0)
        i1 = jax.lax.broadcasted_iota(jnp.int32, (D, D), 1)
        eye = jnp.where(i0 == i1, 1.0, 0.0).astype(f32)
        zp_col = jnp.sum(jnp.broadcast_to(z_p, (D, D)) * eye, axis=1,
                         keepdims=True)
        wcol = jnp.dot(wtopT[...], zp_col, preferred_element_type=f32)
        zsT = z_s.T
        wtsT = jnp.dot(wbotT[...], zsT, preferred_element_type=f32) + wcol
        wtsT_s[...] = wtsT.astype(jnp.bfloat16)

    NC = 4
    CS = VT // NC
    for i in range(NC):
        sl = pl.ds(i * CS, CS)
        argT = jnp.dot(xp[sl], caugT_s[...],
                       preferred_element_type=jnp.float32)
        fT = jnp.exp2(argT.astype(jnp.bfloat16))
        y_ref[0, sl] = jnp.dot(fT, wtsT_s[...],
                               preferred_element_type=jnp.float32)


def kernel(locations, block_subjects, block_tasks, factors_mu, factors_sigma,
           subject_mu, subject_sigma, task_mu, task_sigma, eps_F, eps_P,
           eps_S, W_c, W_w, W_wt):
    xaugT = jnp.concatenate(
        [locations,
         jnp.sum(locations * locations, axis=1, keepdims=True),
         jnp.ones((V, 1), jnp.float32),
         jnp.zeros((V, 3), jnp.float32)], axis=1).astype(jnp.bfloat16)

    wc = W_c.reshape(D, K, 3)
    wc0 = wc[:, :, 0]
    wc1 = wc[:, :, 1]
    wc2 = wc[:, :, 2]
    wtopT = W_wt[:D].T
    wbotT = W_wt[D:].T
    fmu3 = factors_mu[:, None, :]
    fsig3 = factors_sigma[:, None, :]
    smu3 = subject_mu[:, None, :]
    ssig3 = subject_sigma[:, None, :]
    epsF3 = eps_F[:, None, :]
    epsP3 = eps_P[:, None, :]

    nv = pl.cdiv(V, VT)
    grid_spec = pltpu.PrefetchScalarGridSpec(
        num_scalar_prefetch=2,
        grid=(B, nv),
        in_specs=[
            pl.BlockSpec((VT, 8), lambda b, v, s, t: (v, 0)),
            pl.BlockSpec((1, 1, D), lambda b, v, s, t: (s[b], 0, 0)),
            pl.BlockSpec((1, 1, D), lambda b, v, s, t: (s[b], 0, 0)),
            pl.BlockSpec((1, 1, D), lambda b, v, s, t: (s[b], 0, 0)),
            pl.BlockSpec((1, 1, D), lambda b, v, s, t: (s[b], 0, 0)),
            pl.BlockSpec((1, T, D), lambda b, v, s, t: (t[b], 0, 0)),
            pl.BlockSpec((1, T, D), lambda b, v, s, t: (t[b], 0, 0)),
            pl.BlockSpec((1, 1, D), lambda b, v, s, t: (b, 0, 0)),
            pl.BlockSpec((1, 1, D), lambda b, v, s, t: (b, 0, 0)),
            pl.BlockSpec((1, T, D), lambda b, v, s, t: (b, 0, 0)),
            pl.BlockSpec((D, K), lambda b, v, s, t: (0, 0)),
            pl.BlockSpec((D, K), lambda b, v, s, t: (0, 0)),
            pl.BlockSpec((D, K), lambda b, v, s, t: (0, 0)),
            pl.BlockSpec((D, K), lambda b, v, s, t: (0, 0)),
            pl.BlockSpec((K, D), lambda b, v, s, t: (0, 0)),
            pl.BlockSpec((K, D), lambda b, v, s, t: (0, 0)),
        ],
        out_specs=pl.BlockSpec((1, VT, T), lambda b, v, s, t: (b, v, 0)),
        scratch_shapes=[
            pltpu.VMEM((8, K), jnp.bfloat16),
            pltpu.VMEM((K, T), jnp.bfloat16),
        ],
    )
    y = pl.pallas_call(
        _body,
        grid_spec=grid_spec,
        out_shape=jax.ShapeDtypeStruct((B, V, T), jnp.float32),
        compiler_params=pltpu.CompilerParams(
            dimension_semantics=("parallel", "arbitrary"),
        ),
    )(block_subjects, block_tasks, xaugT, fmu3, fsig3, smu3, ssig3,
      task_mu, task_sigma, epsF3, epsP3, eps_S, wc0, wc1, wc2, W_w,
      wtopT, wbotT)
    return jnp.swapaxes(y, 1, 2)
